# Initial kernel scaffold; baseline (speedup 1.0000x reference)
import jax
import jax.numpy as jnp
from jax.experimental import pallas as pl


def kernel(x_user, x_item, edge_index_user_item, edge_index_item_user, l1_ui_Wl, l1_ui_bl, l1_ui_Wr, l1_iu_Wl, l1_iu_bl, l1_iu_Wr, l2_ui_Wl, l2_ui_bl, l2_ui_Wr, l2_iu_Wl, l2_iu_bl, l2_iu_Wr):
    raise NotImplementedError("write your pallas kernel here")


# speedup vs baseline: 2.7120x; 2.7120x over previous
import functools

import jax
import jax.numpy as jnp
from jax import lax
from jax.experimental import pallas as pl
from jax.experimental.pallas import tpu as pltpu
from jax.experimental.pallas import tpu_sc as plsc

NC = 2
NS = 16
CH = 128
PBLK = 4
WBLK = 32


def _sc_mesh():
    return plsc.VectorSubcoreMesh(
        core_axis_name="c", subcore_axis_name="s", num_cores=NC, num_subcores=NS
    )


_SC_PARAMS = pltpu.CompilerParams(use_tc_tiling_on_sc=False)


def _grid_sizes(n_nodes, ep):
    acc_rows = ((n_nodes + NS * 16) // (NS * 16)) * (NS * 16)
    stripe = acc_rows // NS
    last_rows = n_nodes - (NS - 1) * stripe
    assert 0 < last_rows <= stripe and last_rows % 8 == 0
    n_chunks = ep // CH
    kmax = (n_chunks + NS - 1) // NS
    return acc_rows, stripe, last_rows, n_chunks, kmax


def _zchunk(stripe, width):
    for cand in (1, 2, 4, 8):
        if stripe % cand == 0 and (stripe // cand) * width * 4 <= 128 * 1024:
            return cand, stripe // cand
    raise ValueError(stripe)


def _segsum_body(n_nodes, n_chunks, kmax, stripe, last_rows, zrep,
                 xr, srcp, dstp, zrows, out, src2d, dst2d, rowbuf, acc, gsem):
    c = lax.axis_index("c")
    s = lax.axis_index("s")
    zr = zrows.shape[0]
    for pass_ in range(PBLK // NC):
        p = pass_ * NC + c
        for j in range(zrep):
            pltpu.sync_copy(zrows, acc.at[pl.ds(s * stripe + j * zr, zr)])
        plsc.subcore_barrier()

        def chunk_body(k, carry):
            g = s + NS * k

            @pl.when(g < n_chunks)
            def _():
                pltpu.sync_copy(srcp.at[pl.ds(g * CH, CH)], src2d.at[0])
                pltpu.sync_copy(dstp.at[pl.ds(g * CH, CH)], dst2d.at[0])
                for j in range(CH // 16):
                    sl = pl.ds(j * 16, 16)
                    src2d[0, sl] = src2d[0, sl] * PBLK + p
                pltpu.async_copy(xr.at[src2d.at[0]], rowbuf, gsem).wait()
                pltpu.sync_copy(rowbuf, acc.at[dst2d.at[0]], add=True)

            return carry

        lax.fori_loop(0, kmax, chunk_body, 0)
        plsc.subcore_barrier()
        row0 = s * stripe
        col0 = p * WBLK

        @pl.when(s < NS - 1)
        def _():
            pltpu.sync_copy(acc.at[pl.ds(row0, stripe)],
                            out.at[pl.ds(row0, stripe), pl.ds(col0, WBLK)])

        @pl.when(s == NS - 1)
        def _():
            pltpu.sync_copy(acc.at[pl.ds(row0, last_rows)],
                            out.at[pl.ds(row0, last_rows), pl.ds(col0, WBLK)])


def _make_segsum(n_nodes, ep):
    acc_rows, stripe, last_rows, n_chunks, kmax = _grid_sizes(n_nodes, ep)
    zrep, zr = _zchunk(stripe, WBLK)
    body = functools.partial(_segsum_body, n_nodes, n_chunks, kmax,
                             stripe, last_rows, zrep)
    return pl.kernel(
        body,
        out_type=jax.ShapeDtypeStruct((n_nodes, 128), jnp.float32),
        mesh=_sc_mesh(),
        scratch_types=[
            pltpu.VMEM((1, CH), jnp.int32),
            pltpu.VMEM((1, CH), jnp.int32),
            pltpu.VMEM((CH, WBLK), jnp.float32),
            pltpu.VMEM_SHARED((acc_rows, WBLK), jnp.float32),
            pltpu.SemaphoreType.DMA,
        ],
        compiler_params=_SC_PARAMS,
    ), zr


def _counts_body(n_nodes, n_chunks, kmax, stripe, last_rows, zrep, ep,
                 dstp2, ones, zrows, out, dst2d, ones_v, acc, gsem):
    c = lax.axis_index("c")
    s = lax.axis_index("s")
    zr = zrows.shape[0]
    pltpu.sync_copy(ones, ones_v)
    for j in range(zrep):
        pltpu.sync_copy(zrows, acc.at[pl.ds(s * stripe + j * zr, zr)])
    plsc.subcore_barrier()

    def chunk_body(k, carry):
        g = s + NS * k

        @pl.when(g < n_chunks)
        def _():
            pltpu.sync_copy(dstp2.at[pl.ds(c * ep + g * CH, CH)], dst2d.at[0])
            pltpu.sync_copy(ones_v, acc.at[dst2d.at[0]], add=True)

        return carry

    lax.fori_loop(0, kmax, chunk_body, 0)
    plsc.subcore_barrier()
    row0 = s * stripe

    @pl.when(s < NS - 1)
    def _():
        pltpu.sync_copy(acc.at[pl.ds(row0, stripe)],
                        out.at[pl.ds(c * n_nodes + row0, stripe)])

    @pl.when(s == NS - 1)
    def _():
        pltpu.sync_copy(acc.at[pl.ds(row0, last_rows)],
                        out.at[pl.ds(c * n_nodes + row0, last_rows)])


def _make_counts(n_nodes, ep):
    acc_rows, stripe, last_rows, n_chunks, kmax = _grid_sizes(n_nodes, ep)
    zrep, zr = _zchunk(stripe, 16)
    body = functools.partial(_counts_body, n_nodes, n_chunks, kmax,
                             stripe, last_rows, zrep, ep)
    return pl.kernel(
        body,
        out_type=jax.ShapeDtypeStruct((2 * n_nodes, 16), jnp.float32),
        mesh=_sc_mesh(),
        scratch_types=[
            pltpu.VMEM((1, CH), jnp.int32),
            pltpu.VMEM((CH, 16), jnp.float32),
            pltpu.VMEM_SHARED((acc_rows, 16), jnp.float32),
            pltpu.SemaphoreType.DMA,
        ],
        compiler_params=_SC_PARAMS,
    ), zr


def _tc_conv_kernel(relu, sum_ref, cnt_ref, xd_ref, wl_ref, bl_ref, wr_ref, out_ref):
    cnt = jnp.sum(cnt_ref[0], axis=1, keepdims=True) * (1.0 / 16.0)
    mean = sum_ref[...] / jnp.maximum(cnt, 1.0)
    dn = (((1,), (1,)), ((), ()))
    h = (lax.dot_general(mean, wl_ref[...], dn, preferred_element_type=jnp.float32)
         + bl_ref[...]
         + lax.dot_general(xd_ref[...], wr_ref[...], dn,
                           preferred_element_type=jnp.float32))
    if relu:
        h = jnp.maximum(h, 0.0)
    out_ref[...] = h


def _tc_conv(summed, cnts3, etype, x_dst, wl, bl, wr, relu):
    n = summed.shape[0]
    br = 1000 if n % 1000 == 0 else n
    return pl.pallas_call(
        functools.partial(_tc_conv_kernel, relu),
        grid=(n // br,),
        in_specs=[
            pl.BlockSpec((br, 128), lambda i: (i, 0)),
            pl.BlockSpec((1, br, 16), lambda i, t=etype: (t, i, 0)),
            pl.BlockSpec((br, 128), lambda i: (i, 0)),
            pl.BlockSpec((128, 128), lambda i: (0, 0)),
            pl.BlockSpec((1, 128), lambda i: (0, 0)),
            pl.BlockSpec((128, 128), lambda i: (0, 0)),
        ],
        out_specs=pl.BlockSpec((br, 128), lambda i: (i, 0)),
        out_shape=jax.ShapeDtypeStruct((n, 128), jnp.float32),
    )(summed, cnts3, x_dst, wl, bl.reshape(1, 128), wr)


def kernel(x_user, x_item, edge_index_user_item, edge_index_item_user,
           l1_ui_Wl, l1_ui_bl, l1_ui_Wr, l1_iu_Wl, l1_iu_bl, l1_iu_Wr,
           l2_ui_Wl, l2_ui_bl, l2_ui_Wr, l2_iu_Wl, l2_iu_bl, l2_iu_Wr):
    n = x_user.shape[0]
    e = edge_index_user_item.shape[1]
    ep = ((e + CH - 1) // CH) * CH
    padn = ep - e
    i32 = jnp.int32

    def pad_edges(ei):
        src = jnp.concatenate([ei[0].astype(i32), jnp.zeros((padn,), i32)])
        dst = jnp.concatenate([ei[1].astype(i32), jnp.full((padn,), n, i32)])
        return src, dst

    src_ui, dst_ui = pad_edges(edge_index_user_item)
    src_iu, dst_iu = pad_edges(edge_index_item_user)

    segsum, zr = _make_segsum(n, ep)
    counts, zrc = _make_counts(n, ep)

    zrows = jnp.zeros((zr, WBLK), jnp.float32)
    zrows16 = jnp.zeros((zrc, 16), jnp.float32)
    ones16 = jnp.ones((CH, 16), jnp.float32)

    dstp2 = jnp.concatenate([dst_ui, dst_iu])
    cnts3 = counts(dstp2, ones16, zrows16).reshape(2, n, 16)

    sum1_item = segsum(x_user.reshape(PBLK * n, WBLK), src_ui, dst_ui, zrows)
    sum1_user = segsum(x_item.reshape(PBLK * n, WBLK), src_iu, dst_iu, zrows)
    h1_item = _tc_conv(sum1_item, cnts3, 0, x_item, l1_ui_Wl, l1_ui_bl, l1_ui_Wr, True)
    h1_user = _tc_conv(sum1_user, cnts3, 1, x_user, l1_iu_Wl, l1_iu_bl, l1_iu_Wr, True)

    sum2_item = segsum(h1_user.reshape(PBLK * n, WBLK), src_ui, dst_ui, zrows)
    sum2_user = segsum(h1_item.reshape(PBLK * n, WBLK), src_iu, dst_iu, zrows)
    h2_item = _tc_conv(sum2_item, cnts3, 0, h1_item, l2_ui_Wl, l2_ui_bl, l2_ui_Wr, False)
    h2_user = _tc_conv(sum2_user, cnts3, 1, h1_user, l2_iu_Wl, l2_iu_bl, l2_iu_Wr, False)
    return (h2_user, h2_item)


# speedup vs baseline: 6.1648x; 2.2732x over previous
import functools

import jax
import jax.numpy as jnp
from jax import lax
from jax.experimental import pallas as pl
from jax.experimental.pallas import tpu as pltpu
from jax.experimental.pallas import tpu_sc as plsc

NC = 2
NS = 16
CH = 128
PBLK = 4
WBLK = 32
NBUF = 4


def _sc_mesh():
    return plsc.VectorSubcoreMesh(
        core_axis_name="c", subcore_axis_name="s", num_cores=NC, num_subcores=NS
    )


_SC_PARAMS = pltpu.CompilerParams(use_tc_tiling_on_sc=False)


def _grid_sizes(n_nodes, n_chunks):
    acc_rows = ((n_nodes + NS * 16) // (NS * 16)) * (NS * 16)
    stripe = acc_rows // NS
    last_rows = n_nodes - (NS - 1) * stripe
    assert 0 < last_rows <= stripe and last_rows % 8 == 0
    km = n_chunks // NS
    assert km % (2 * NBUF) == 0
    return acc_rows, stripe, last_rows, km


def _zchunk(stripe, width):
    for cand in (1, 2, 4, 8):
        if stripe % cand == 0 and (stripe // cand) * width * 4 <= 128 * 1024:
            return cand, stripe // cand
    raise ValueError(stripe)


def _segsum_body(n_nodes, km, stripe, last_rows, zrep,
                 xr, srcp4, dstp2d, zrows, out,
                 src_stage, dst_stage, rowbuf, acc, *gsems):
    c = lax.axis_index("c")
    s = lax.axis_index("s")
    zr = zrows.shape[0]
    ngroups = km // NBUF

    for pass_ in range(PBLK // NC):
        p = pass_ * NC + c
        for j in range(zrep):
            pltpu.sync_copy(zrows, acc.at[pl.ds(s * stripe + j * zr, zr)])
        plsc.subcore_barrier()
        pass_base = (p * NS + s) * km
        dst_base = s * km

        def stage(g, par):
            pltpu.sync_copy(srcp4.at[pl.ds(pass_base + g * NBUF, NBUF)],
                            src_stage.at[par])
            pltpu.sync_copy(dstp2d.at[pl.ds(dst_base + g * NBUF, NBUF)],
                            dst_stage.at[par])

        stage(0, 0)

        def gbody(gg, carry):
            for half in range(2):
                g = 2 * gg + half
                par, npar = half, 1 - half
                descs = [
                    pltpu.async_copy(xr.at[src_stage.at[par, b]],
                                     rowbuf.at[b], gsems[b])
                    for b in range(NBUF)
                ]

                @pl.when(g + 1 < ngroups)
                def _():
                    stage(g + 1, npar)

                for b in range(NBUF):
                    descs[b].wait()
                    pltpu.sync_copy(rowbuf.at[b],
                                    acc.at[dst_stage.at[par, b]], add=True)
            return carry

        lax.fori_loop(0, ngroups // 2, gbody, 0)
        plsc.subcore_barrier()
        row0 = s * stripe
        col0 = p * WBLK

        @pl.when(s < NS - 1)
        def _():
            pltpu.sync_copy(acc.at[pl.ds(row0, stripe)],
                            out.at[pl.ds(row0, stripe), pl.ds(col0, WBLK)])

        @pl.when(s == NS - 1)
        def _():
            pltpu.sync_copy(acc.at[pl.ds(row0, last_rows)],
                            out.at[pl.ds(row0, last_rows), pl.ds(col0, WBLK)])


def _make_segsum(n_nodes, n_chunks):
    acc_rows, stripe, last_rows, km = _grid_sizes(n_nodes, n_chunks)
    zrep, zr = _zchunk(stripe, WBLK)
    body = functools.partial(_segsum_body, n_nodes, km, stripe, last_rows, zrep)
    return pl.kernel(
        body,
        out_type=jax.ShapeDtypeStruct((n_nodes, 128), jnp.float32),
        mesh=_sc_mesh(),
        scratch_types=[
            pltpu.VMEM((2, NBUF, CH), jnp.int32),
            pltpu.VMEM((2, NBUF, CH), jnp.int32),
            pltpu.VMEM((NBUF, CH, WBLK), jnp.float32),
            pltpu.VMEM_SHARED((acc_rows, WBLK), jnp.float32),
        ] + [pltpu.SemaphoreType.DMA] * NBUF,
        compiler_params=_SC_PARAMS,
    ), zr


def _counts_body(n_nodes, km, stripe, last_rows, zrep,
                 dstc2d, ones, zrows, out, dst_stage, ones_v, zbuf, acc):
    c = lax.axis_index("c")
    s = lax.axis_index("s")
    zr = zrows.shape[0]
    pltpu.sync_copy(ones, ones_v)
    pltpu.sync_copy(zrows, zbuf)
    for j in range(zrep):
        pltpu.sync_copy(zbuf, acc.at[pl.ds(s * stripe + j * zr, zr)])
    plsc.subcore_barrier()
    dst_base = (c * NS + s) * km

    def gbody(g, carry):
        pltpu.sync_copy(dstc2d.at[pl.ds(dst_base + g * NBUF, NBUF)], dst_stage)
        for b in range(NBUF):
            pltpu.sync_copy(ones_v, acc.at[dst_stage.at[b]], add=True)
        return carry

    lax.fori_loop(0, km // NBUF, gbody, 0)
    plsc.subcore_barrier()
    row0 = s * stripe

    @pl.when(s < NS - 1)
    def _():
        pltpu.sync_copy(acc.at[pl.ds(row0, stripe)],
                        out.at[pl.ds(c * n_nodes + row0, stripe)])

    @pl.when(s == NS - 1)
    def _():
        pltpu.sync_copy(acc.at[pl.ds(row0, last_rows)],
                        out.at[pl.ds(c * n_nodes + row0, last_rows)])


def _make_counts(n_nodes, n_chunks):
    acc_rows, stripe, last_rows, km = _grid_sizes(n_nodes, n_chunks)
    zrep, zr = _zchunk(stripe, 16)
    body = functools.partial(_counts_body, n_nodes, km, stripe, last_rows, zrep)
    return pl.kernel(
        body,
        out_type=jax.ShapeDtypeStruct((2 * n_nodes, 16), jnp.float32),
        mesh=_sc_mesh(),
        scratch_types=[
            pltpu.VMEM((NBUF, CH), jnp.int32),
            pltpu.VMEM((CH, 16), jnp.float32),
            pltpu.VMEM((zr, 16), jnp.float32),
            pltpu.VMEM_SHARED((acc_rows, 16), jnp.float32),
        ],
        compiler_params=_SC_PARAMS,
    ), zr


def _tc_conv_kernel(relu, sum_ref, cnt_ref, xd_ref, wl_ref, bl_ref, wr_ref, out_ref):
    cnt = jnp.sum(cnt_ref[0], axis=1, keepdims=True) * (1.0 / 16.0)
    mean = sum_ref[...] / jnp.maximum(cnt, 1.0)
    dn = (((1,), (1,)), ((), ()))
    h = (lax.dot_general(mean, wl_ref[...], dn, preferred_element_type=jnp.float32)
         + bl_ref[...]
         + lax.dot_general(xd_ref[...], wr_ref[...], dn,
                           preferred_element_type=jnp.float32))
    if relu:
        h = jnp.maximum(h, 0.0)
    out_ref[...] = h


def _tc_conv(summed, cnts3, etype, x_dst, wl, bl, wr, relu):
    n = summed.shape[0]
    br = 1000 if n % 1000 == 0 else n
    return pl.pallas_call(
        functools.partial(_tc_conv_kernel, relu),
        grid=(n // br,),
        in_specs=[
            pl.BlockSpec((br, 128), lambda i: (i, 0)),
            pl.BlockSpec((1, br, 16), lambda i, t=etype: (t, i, 0)),
            pl.BlockSpec((br, 128), lambda i: (i, 0)),
            pl.BlockSpec((128, 128), lambda i: (0, 0)),
            pl.BlockSpec((1, 128), lambda i: (0, 0)),
            pl.BlockSpec((128, 128), lambda i: (0, 0)),
        ],
        out_specs=pl.BlockSpec((br, 128), lambda i: (i, 0)),
        out_shape=jax.ShapeDtypeStruct((n, 128), jnp.float32),
    )(summed, cnts3, x_dst, wl, bl.reshape(1, 128), wr)


def kernel(x_user, x_item, edge_index_user_item, edge_index_item_user,
           l1_ui_Wl, l1_ui_bl, l1_ui_Wr, l1_iu_Wl, l1_iu_bl, l1_iu_Wr,
           l2_ui_Wl, l2_ui_bl, l2_ui_Wr, l2_iu_Wl, l2_iu_bl, l2_iu_Wr):
    n = x_user.shape[0]
    e = edge_index_user_item.shape[1]
    gq = NS * 2 * NBUF
    g_chunks = ((e + CH - 1) // CH + gq - 1) // gq * gq
    ep = g_chunks * CH
    padn = ep - e
    i32 = jnp.int32
    trash = ((n + NS * 16) // (NS * 16)) * (NS * 16) - n
    pad_src = (jnp.arange(padn, dtype=i32) * 97) % n
    pad_dst = n + (jnp.arange(padn, dtype=i32) % max(trash, 1))

    def prep_edges(ei):
        src = jnp.concatenate([ei[0].astype(i32), pad_src])
        dst = jnp.concatenate([ei[1].astype(i32), pad_dst])
        srcr = src.reshape(g_chunks, CH)
        src4 = srcr[None] * PBLK + jnp.arange(PBLK, dtype=i32)[:, None, None]
        return src4.reshape(PBLK * g_chunks, CH), dst.reshape(g_chunks, CH)

    src4_ui, dst2d_ui = prep_edges(edge_index_user_item)
    src4_iu, dst2d_iu = prep_edges(edge_index_item_user)

    segsum, zr = _make_segsum(n, g_chunks)
    counts, zrc = _make_counts(n, g_chunks)

    zrows = jnp.zeros((zr, WBLK), jnp.float32)
    zrows16 = jnp.zeros((zrc, 16), jnp.float32)
    ones16 = jnp.ones((CH, 16), jnp.float32)

    dstc2d = jnp.concatenate([dst2d_ui, dst2d_iu])
    cnts3 = counts(dstc2d, ones16, zrows16).reshape(2, n, 16)

    sum1_item = segsum(x_user.reshape(PBLK * n, WBLK), src4_ui, dst2d_ui, zrows)
    sum1_user = segsum(x_item.reshape(PBLK * n, WBLK), src4_iu, dst2d_iu, zrows)
    h1_item = _tc_conv(sum1_item, cnts3, 0, x_item, l1_ui_Wl, l1_ui_bl, l1_ui_Wr, True)
    h1_user = _tc_conv(sum1_user, cnts3, 1, x_user, l1_iu_Wl, l1_iu_bl, l1_iu_Wr, True)

    sum2_item = segsum(h1_user.reshape(PBLK * n, WBLK), src4_ui, dst2d_ui, zrows)
    sum2_user = segsum(h1_item.reshape(PBLK * n, WBLK), src4_iu, dst2d_iu, zrows)
    h2_item = _tc_conv(sum2_item, cnts3, 0, h1_item, l2_ui_Wl, l2_ui_bl, l2_ui_Wr, False)
    h2_user = _tc_conv(sum2_user, cnts3, 1, h1_user, l2_iu_Wl, l2_iu_bl, l2_iu_Wr, False)
    return (h2_user, h2_item)


# speedup vs baseline: 6.3381x; 1.0281x over previous
import functools

import jax
import jax.numpy as jnp
from jax import lax
from jax.experimental import pallas as pl
from jax.experimental.pallas import tpu as pltpu
from jax.experimental.pallas import tpu_sc as plsc

NC = 2
NS = 16
CH = 128
PBLK = 4
WBLK = 32
NBUF = 3


def _sc_mesh():
    return plsc.VectorSubcoreMesh(
        core_axis_name="c", subcore_axis_name="s", num_cores=NC, num_subcores=NS
    )


_SC_PARAMS = pltpu.CompilerParams(use_tc_tiling_on_sc=False)


def _grid_sizes(n_nodes, n_chunks):
    acc_rows = ((n_nodes + NS * 16) // (NS * 16)) * (NS * 16)
    stripe = acc_rows // NS
    last_rows = n_nodes - (NS - 1) * stripe
    assert 0 < last_rows <= stripe and last_rows % 8 == 0
    km = n_chunks // NS
    assert km % (2 * NBUF) == 0
    return acc_rows, stripe, last_rows, km


def _zchunk(stripe, width):
    for cand in (1, 2, 4, 8):
        if stripe % cand == 0 and (stripe // cand) * width * 4 <= 128 * 1024:
            return cand, stripe // cand
    raise ValueError(stripe)


def _segsum_body(n_nodes, km, stripe, last_rows, zrep,
                 xr, srcp4, dstp2d, zrows, out,
                 src_stage, dst_stage, rowbuf, acc, *gsems):
    c = lax.axis_index("c")
    s = lax.axis_index("s")
    zr = zrows.shape[0]
    ngroups = km // NBUF

    for pass_ in range(PBLK // NC):
        p = pass_ * NC + c
        for j in range(zrep):
            pltpu.sync_copy(zrows, acc.at[pl.ds(s * stripe + j * zr, zr)])
        plsc.subcore_barrier()
        pass_base = (p * NS + s) * km
        dst_base = s * km

        def stage(g, par):
            pltpu.sync_copy(srcp4.at[pl.ds(pass_base + g * NBUF, NBUF)],
                            src_stage.at[par])
            pltpu.sync_copy(dstp2d.at[pl.ds(dst_base + g * NBUF, NBUF)],
                            dst_stage.at[par])

        def issue(par):
            for b in range(NBUF):
                pltpu.async_copy(xr.at[src_stage.at[par, b]],
                                 rowbuf.at[par * NBUF + b],
                                 gsems[par * NBUF + b])

        stage(0, 0)
        stage(1, 1)
        issue(0)

        def gbody(gg, carry):
            for par in range(2):
                g = 2 * gg + par
                npar = 1 - par

                @pl.when(g + 1 < ngroups)
                def _():
                    issue(npar)

                for b in range(NBUF):
                    pltpu.make_async_copy(
                        xr.at[src_stage.at[par, b]],
                        rowbuf.at[par * NBUF + b],
                        gsems[par * NBUF + b]).wait()
                    pltpu.sync_copy(rowbuf.at[par * NBUF + b],
                                    acc.at[dst_stage.at[par, b]], add=True)

                @pl.when(g + 2 < ngroups)
                def _():
                    stage(g + 2, par)
            return carry

        lax.fori_loop(0, ngroups // 2, gbody, 0)
        plsc.subcore_barrier()
        row0 = s * stripe
        col0 = p * WBLK

        @pl.when(s < NS - 1)
        def _():
            pltpu.sync_copy(acc.at[pl.ds(row0, stripe)],
                            out.at[pl.ds(row0, stripe), pl.ds(col0, WBLK)])

        @pl.when(s == NS - 1)
        def _():
            pltpu.sync_copy(acc.at[pl.ds(row0, last_rows)],
                            out.at[pl.ds(row0, last_rows), pl.ds(col0, WBLK)])


def _make_segsum(n_nodes, n_chunks):
    acc_rows, stripe, last_rows, km = _grid_sizes(n_nodes, n_chunks)
    zrep, zr = _zchunk(stripe, WBLK)
    body = functools.partial(_segsum_body, n_nodes, km, stripe, last_rows, zrep)
    return pl.kernel(
        body,
        out_type=jax.ShapeDtypeStruct((n_nodes, 128), jnp.float32),
        mesh=_sc_mesh(),
        scratch_types=[
            pltpu.VMEM((2, NBUF, CH), jnp.int32),
            pltpu.VMEM((2, NBUF, CH), jnp.int32),
            pltpu.VMEM((2 * NBUF, CH, WBLK), jnp.float32),
            pltpu.VMEM_SHARED((acc_rows, WBLK), jnp.float32),
        ] + [pltpu.SemaphoreType.DMA] * (2 * NBUF),
        compiler_params=_SC_PARAMS,
    ), zr


def _counts_body(n_nodes, km, stripe, last_rows, zrep,
                 dstc2d, ones, zrows, out, dst_stage, ones_v, zbuf, acc):
    c = lax.axis_index("c")
    s = lax.axis_index("s")
    zr = zrows.shape[0]
    pltpu.sync_copy(ones, ones_v)
    pltpu.sync_copy(zrows, zbuf)
    for j in range(zrep):
        pltpu.sync_copy(zbuf, acc.at[pl.ds(s * stripe + j * zr, zr)])
    plsc.subcore_barrier()
    dst_base = (c * NS + s) * km

    def gbody(g, carry):
        pltpu.sync_copy(dstc2d.at[pl.ds(dst_base + g * NBUF, NBUF)], dst_stage)
        for b in range(NBUF):
            pltpu.sync_copy(ones_v, acc.at[dst_stage.at[b]], add=True)
        return carry

    lax.fori_loop(0, km // NBUF, gbody, 0)
    plsc.subcore_barrier()
    row0 = s * stripe

    @pl.when(s < NS - 1)
    def _():
        pltpu.sync_copy(acc.at[pl.ds(row0, stripe)],
                        out.at[pl.ds(c * n_nodes + row0, stripe)])

    @pl.when(s == NS - 1)
    def _():
        pltpu.sync_copy(acc.at[pl.ds(row0, last_rows)],
                        out.at[pl.ds(c * n_nodes + row0, last_rows)])


def _make_counts(n_nodes, n_chunks):
    acc_rows, stripe, last_rows, km = _grid_sizes(n_nodes, n_chunks)
    zrep, zr = _zchunk(stripe, 16)
    body = functools.partial(_counts_body, n_nodes, km, stripe, last_rows, zrep)
    return pl.kernel(
        body,
        out_type=jax.ShapeDtypeStruct((2 * n_nodes, 16), jnp.float32),
        mesh=_sc_mesh(),
        scratch_types=[
            pltpu.VMEM((NBUF, CH), jnp.int32),
            pltpu.VMEM((CH, 16), jnp.float32),
            pltpu.VMEM((zr, 16), jnp.float32),
            pltpu.VMEM_SHARED((acc_rows, 16), jnp.float32),
        ],
        compiler_params=_SC_PARAMS,
    ), zr


def _tc_conv_kernel(relu, sum_ref, cnt_ref, xd_ref, wl_ref, bl_ref, wr_ref, out_ref):
    cnt = jnp.sum(cnt_ref[0], axis=1, keepdims=True) * (1.0 / 16.0)
    mean = sum_ref[...] / jnp.maximum(cnt, 1.0)
    dn = (((1,), (1,)), ((), ()))
    h = (lax.dot_general(mean, wl_ref[...], dn, preferred_element_type=jnp.float32)
         + bl_ref[...]
         + lax.dot_general(xd_ref[...], wr_ref[...], dn,
                           preferred_element_type=jnp.float32))
    if relu:
        h = jnp.maximum(h, 0.0)
    out_ref[...] = h


def _tc_conv(summed, cnts3, etype, x_dst, wl, bl, wr, relu):
    n = summed.shape[0]
    br = 1000 if n % 1000 == 0 else n
    return pl.pallas_call(
        functools.partial(_tc_conv_kernel, relu),
        grid=(n // br,),
        in_specs=[
            pl.BlockSpec((br, 128), lambda i: (i, 0)),
            pl.BlockSpec((1, br, 16), lambda i, t=etype: (t, i, 0)),
            pl.BlockSpec((br, 128), lambda i: (i, 0)),
            pl.BlockSpec((128, 128), lambda i: (0, 0)),
            pl.BlockSpec((1, 128), lambda i: (0, 0)),
            pl.BlockSpec((128, 128), lambda i: (0, 0)),
        ],
        out_specs=pl.BlockSpec((br, 128), lambda i: (i, 0)),
        out_shape=jax.ShapeDtypeStruct((n, 128), jnp.float32),
    )(summed, cnts3, x_dst, wl, bl.reshape(1, 128), wr)


def kernel(x_user, x_item, edge_index_user_item, edge_index_item_user,
           l1_ui_Wl, l1_ui_bl, l1_ui_Wr, l1_iu_Wl, l1_iu_bl, l1_iu_Wr,
           l2_ui_Wl, l2_ui_bl, l2_ui_Wr, l2_iu_Wl, l2_iu_bl, l2_iu_Wr):
    n = x_user.shape[0]
    e = edge_index_user_item.shape[1]
    gq = NS * 2 * NBUF
    g_chunks = ((e + CH - 1) // CH + gq - 1) // gq * gq
    ep = g_chunks * CH
    padn = ep - e
    i32 = jnp.int32
    trash = ((n + NS * 16) // (NS * 16)) * (NS * 16) - n
    pad_src = (jnp.arange(padn, dtype=i32) * 97) % n
    pad_dst = n + (jnp.arange(padn, dtype=i32) % max(trash, 1))

    def prep_edges(ei):
        src = jnp.concatenate([ei[0].astype(i32), pad_src])
        dst = jnp.concatenate([ei[1].astype(i32), pad_dst])
        srcr = src.reshape(g_chunks, CH)
        src4 = srcr[None] * PBLK + jnp.arange(PBLK, dtype=i32)[:, None, None]
        return src4.reshape(PBLK * g_chunks, CH), dst.reshape(g_chunks, CH)

    src4_ui, dst2d_ui = prep_edges(edge_index_user_item)
    src4_iu, dst2d_iu = prep_edges(edge_index_item_user)

    segsum, zr = _make_segsum(n, g_chunks)
    counts, zrc = _make_counts(n, g_chunks)

    zrows = jnp.zeros((zr, WBLK), jnp.float32)
    zrows16 = jnp.zeros((zrc, 16), jnp.float32)
    ones16 = jnp.ones((CH, 16), jnp.float32)

    dstc2d = jnp.concatenate([dst2d_ui, dst2d_iu])
    cnts3 = counts(dstc2d, ones16, zrows16).reshape(2, n, 16)

    sum1_item = segsum(x_user.reshape(PBLK * n, WBLK), src4_ui, dst2d_ui, zrows)
    sum1_user = segsum(x_item.reshape(PBLK * n, WBLK), src4_iu, dst2d_iu, zrows)
    h1_item = _tc_conv(sum1_item, cnts3, 0, x_item, l1_ui_Wl, l1_ui_bl, l1_ui_Wr, True)
    h1_user = _tc_conv(sum1_user, cnts3, 1, x_user, l1_iu_Wl, l1_iu_bl, l1_iu_Wr, True)

    sum2_item = segsum(h1_user.reshape(PBLK * n, WBLK), src4_ui, dst2d_ui, zrows)
    sum2_user = segsum(h1_item.reshape(PBLK * n, WBLK), src4_iu, dst2d_iu, zrows)
    h2_item = _tc_conv(sum2_item, cnts3, 0, h1_item, l2_ui_Wl, l2_ui_bl, l2_ui_Wr, False)
    h2_user = _tc_conv(sum2_user, cnts3, 1, h1_user, l2_iu_Wl, l2_iu_bl, l2_iu_Wr, False)
    return (h2_user, h2_item)


# speedup vs baseline: 6.6766x; 1.0534x over previous
import functools

import jax
import jax.numpy as jnp
from jax import lax
from jax.experimental import pallas as pl
from jax.experimental.pallas import tpu as pltpu
from jax.experimental.pallas import tpu_sc as plsc

NC = 2
NS = 16
CH = 128
PBLK = 4
WBLK = 32
NBUF = 3


def _sc_mesh():
    return plsc.VectorSubcoreMesh(
        core_axis_name="c", subcore_axis_name="s", num_cores=NC, num_subcores=NS
    )


_SC_PARAMS = pltpu.CompilerParams(use_tc_tiling_on_sc=False)


def _grid_sizes(n_nodes, n_chunks):
    acc_rows = ((n_nodes + NS * 16) // (NS * 16)) * (NS * 16)
    stripe = acc_rows // NS
    last_rows = n_nodes - (NS - 1) * stripe
    assert 0 < last_rows <= stripe and last_rows % 8 == 0
    km = n_chunks // NS
    assert km % (2 * NBUF) == 0
    return acc_rows, stripe, last_rows, km


def _zchunk(stripe, width):
    for cand in (1, 2, 4, 8):
        if stripe % cand == 0 and (stripe // cand) * width * 4 <= 128 * 1024:
            return cand, stripe // cand
    raise ValueError(stripe)


def _segsum_body(n_nodes, km, stripe, last_rows, zrep,
                 xr, srcp4, dstp2d, zrows, out,
                 src_stage, dst_stage, rowbuf, acc, *sems):
    gsems = sems[:2 * NBUF]
    ssems = sems[2 * NBUF:]
    c = lax.axis_index("c")
    s = lax.axis_index("s")
    zr = zrows.shape[0]
    ngroups = km // NBUF

    for pass_ in range(PBLK // NC):
        p = pass_ * NC + c
        for j in range(zrep):
            pltpu.sync_copy(zrows, acc.at[pl.ds(s * stripe + j * zr, zr)])
        plsc.subcore_barrier()
        pass_base = (p * NS + s) * km
        dst_base = s * km

        def stage(g, par):
            pltpu.sync_copy(srcp4.at[pl.ds(pass_base + g * NBUF, NBUF)],
                            src_stage.at[par])
            pltpu.sync_copy(dstp2d.at[pl.ds(dst_base + g * NBUF, NBUF)],
                            dst_stage.at[par])

        def issue_gather(par, b):
            pltpu.async_copy(xr.at[src_stage.at[par, b]], rowbuf.at[par, b],
                             gsems[par * NBUF + b])

        def wait_gather(par, b):
            pltpu.make_async_copy(xr.at[src_stage.at[par, b]],
                                  rowbuf.at[par, b],
                                  gsems[par * NBUF + b]).wait()

        def issue_scatter(par, b):
            pltpu.async_copy(rowbuf.at[par, b], acc.at[dst_stage.at[par, b]],
                             ssems[par * NBUF + b], add=True)

        def wait_scatter(par, b):
            pltpu.make_async_copy(rowbuf.at[par, b],
                                  acc.at[dst_stage.at[par, b]],
                                  ssems[par * NBUF + b]).wait()

        stage(0, 0)
        for b in range(NBUF):
            issue_gather(0, b)

        def gbody(gg, carry):
            for par in range(2):
                u = 2 * gg + par
                npar = 1 - par

                @pl.when(u > 0)
                def _():
                    for b in range(NBUF):
                        wait_scatter(npar, b)

                @pl.when(u + 1 < ngroups)
                def _():
                    stage(u + 1, npar)
                    for b in range(NBUF):
                        issue_gather(npar, b)

                for b in range(NBUF):
                    wait_gather(par, b)
                    issue_scatter(par, b)
            return carry

        lax.fori_loop(0, ngroups // 2, gbody, 0)
        for b in range(NBUF):
            wait_scatter(1, b)
        plsc.subcore_barrier()
        row0 = s * stripe
        col0 = p * WBLK

        @pl.when(s < NS - 1)
        def _():
            pltpu.sync_copy(acc.at[pl.ds(row0, stripe)],
                            out.at[pl.ds(row0, stripe), pl.ds(col0, WBLK)])

        @pl.when(s == NS - 1)
        def _():
            pltpu.sync_copy(acc.at[pl.ds(row0, last_rows)],
                            out.at[pl.ds(row0, last_rows), pl.ds(col0, WBLK)])


def _make_segsum(n_nodes, n_chunks):
    acc_rows, stripe, last_rows, km = _grid_sizes(n_nodes, n_chunks)
    zrep, zr = _zchunk(stripe, WBLK)
    body = functools.partial(_segsum_body, n_nodes, km, stripe, last_rows, zrep)
    return pl.kernel(
        body,
        out_type=jax.ShapeDtypeStruct((n_nodes, 128), jnp.float32),
        mesh=_sc_mesh(),
        scratch_types=[
            pltpu.VMEM((2, NBUF, CH), jnp.int32),
            pltpu.VMEM((2, NBUF, CH), jnp.int32),
            pltpu.VMEM((2, NBUF, CH, WBLK), jnp.float32),
            pltpu.VMEM_SHARED((acc_rows, WBLK), jnp.float32),
        ] + [pltpu.SemaphoreType.DMA] * (4 * NBUF),
        compiler_params=_SC_PARAMS,
    ), zr


def _counts_body(n_nodes, km, stripe, last_rows, zrep,
                 dstc2d, ones, zrows, out, dst_stage, ones_v, zbuf, acc):
    c = lax.axis_index("c")
    s = lax.axis_index("s")
    zr = zrows.shape[0]
    pltpu.sync_copy(ones, ones_v)
    pltpu.sync_copy(zrows, zbuf)
    for j in range(zrep):
        pltpu.sync_copy(zbuf, acc.at[pl.ds(s * stripe + j * zr, zr)])
    plsc.subcore_barrier()
    dst_base = (c * NS + s) * km

    def gbody(g, carry):
        pltpu.sync_copy(dstc2d.at[pl.ds(dst_base + g * NBUF, NBUF)], dst_stage)
        for b in range(NBUF):
            pltpu.sync_copy(ones_v.at[b], acc.at[dst_stage.at[b]], add=True)
        return carry

    lax.fori_loop(0, km // NBUF, gbody, 0)
    plsc.subcore_barrier()
    row0 = s * stripe

    @pl.when(s < NS - 1)
    def _():
        pltpu.sync_copy(acc.at[pl.ds(row0, stripe)],
                        out.at[pl.ds(c * n_nodes + row0, stripe)])

    @pl.when(s == NS - 1)
    def _():
        pltpu.sync_copy(acc.at[pl.ds(row0, last_rows)],
                        out.at[pl.ds(c * n_nodes + row0, last_rows)])


def _make_counts(n_nodes, n_chunks):
    acc_rows, stripe, last_rows, km = _grid_sizes(n_nodes, n_chunks)
    zrep, zr = _zchunk(stripe, 16)
    body = functools.partial(_counts_body, n_nodes, km, stripe, last_rows, zrep)
    return pl.kernel(
        body,
        out_type=jax.ShapeDtypeStruct((2 * n_nodes, 16), jnp.float32),
        mesh=_sc_mesh(),
        scratch_types=[
            pltpu.VMEM((NBUF, CH), jnp.int32),
            pltpu.VMEM((NBUF, CH, 16), jnp.float32),
            pltpu.VMEM((zr, 16), jnp.float32),
            pltpu.VMEM_SHARED((acc_rows, 16), jnp.float32),
        ],
        compiler_params=_SC_PARAMS,
    ), zr


def _tc_conv_kernel(relu, sum_ref, cnt_ref, xd_ref, wl_ref, bl_ref, wr_ref, out_ref):
    cnt = jnp.sum(cnt_ref[0], axis=1, keepdims=True) * (1.0 / 16.0)
    mean = sum_ref[...] / jnp.maximum(cnt, 1.0)
    dn = (((1,), (1,)), ((), ()))
    h = (lax.dot_general(mean, wl_ref[...], dn, preferred_element_type=jnp.float32)
         + bl_ref[...]
         + lax.dot_general(xd_ref[...], wr_ref[...], dn,
                           preferred_element_type=jnp.float32))
    if relu:
        h = jnp.maximum(h, 0.0)
    out_ref[...] = h


def _tc_conv(summed, cnts3, etype, x_dst, wl, bl, wr, relu):
    n = summed.shape[0]
    br = 1000 if n % 1000 == 0 else n
    return pl.pallas_call(
        functools.partial(_tc_conv_kernel, relu),
        grid=(n // br,),
        in_specs=[
            pl.BlockSpec((br, 128), lambda i: (i, 0)),
            pl.BlockSpec((1, br, 16), lambda i, t=etype: (t, i, 0)),
            pl.BlockSpec((br, 128), lambda i: (i, 0)),
            pl.BlockSpec((128, 128), lambda i: (0, 0)),
            pl.BlockSpec((1, 128), lambda i: (0, 0)),
            pl.BlockSpec((128, 128), lambda i: (0, 0)),
        ],
        out_specs=pl.BlockSpec((br, 128), lambda i: (i, 0)),
        out_shape=jax.ShapeDtypeStruct((n, 128), jnp.float32),
    )(summed, cnts3, x_dst, wl, bl.reshape(1, 128), wr)


def kernel(x_user, x_item, edge_index_user_item, edge_index_item_user,
           l1_ui_Wl, l1_ui_bl, l1_ui_Wr, l1_iu_Wl, l1_iu_bl, l1_iu_Wr,
           l2_ui_Wl, l2_ui_bl, l2_ui_Wr, l2_iu_Wl, l2_iu_bl, l2_iu_Wr):
    n = x_user.shape[0]
    e = edge_index_user_item.shape[1]
    gq = NS * 2 * NBUF
    g_chunks = ((e + CH - 1) // CH + gq - 1) // gq * gq
    ep = g_chunks * CH
    padn = ep - e
    i32 = jnp.int32
    trash = ((n + NS * 16) // (NS * 16)) * (NS * 16) - n
    pad_src = (jnp.arange(padn, dtype=i32) * 97) % n
    pad_dst = n + (jnp.arange(padn, dtype=i32) % max(trash, 1))

    def prep_edges(ei):
        src = jnp.concatenate([ei[0].astype(i32), pad_src])
        dst = jnp.concatenate([ei[1].astype(i32), pad_dst])
        srcr = src.reshape(g_chunks, CH)
        src4 = srcr[None] * PBLK + jnp.arange(PBLK, dtype=i32)[:, None, None]
        return src4.reshape(PBLK * g_chunks, CH), dst.reshape(g_chunks, CH)

    src4_ui, dst2d_ui = prep_edges(edge_index_user_item)
    src4_iu, dst2d_iu = prep_edges(edge_index_item_user)

    segsum, zr = _make_segsum(n, g_chunks)
    counts, zrc = _make_counts(n, g_chunks)

    zrows = jnp.zeros((zr, WBLK), jnp.float32)
    zrows16 = jnp.zeros((zrc, 16), jnp.float32)
    ones16 = jnp.ones((NBUF, CH, 16), jnp.float32)

    dstc2d = jnp.concatenate([dst2d_ui, dst2d_iu])
    cnts3 = counts(dstc2d, ones16, zrows16).reshape(2, n, 16)

    sum1_item = segsum(x_user.reshape(PBLK * n, WBLK), src4_ui, dst2d_ui, zrows)
    sum1_user = segsum(x_item.reshape(PBLK * n, WBLK), src4_iu, dst2d_iu, zrows)
    h1_item = _tc_conv(sum1_item, cnts3, 0, x_item, l1_ui_Wl, l1_ui_bl, l1_ui_Wr, True)
    h1_user = _tc_conv(sum1_user, cnts3, 1, x_user, l1_iu_Wl, l1_iu_bl, l1_iu_Wr, True)

    sum2_item = segsum(h1_user.reshape(PBLK * n, WBLK), src4_ui, dst2d_ui, zrows)
    sum2_user = segsum(h1_item.reshape(PBLK * n, WBLK), src4_iu, dst2d_iu, zrows)
    h2_item = _tc_conv(sum2_item, cnts3, 0, h1_item, l2_ui_Wl, l2_ui_bl, l2_ui_Wr, False)
    h2_user = _tc_conv(sum2_user, cnts3, 1, h1_user, l2_iu_Wl, l2_iu_bl, l2_iu_Wr, False)
    return (h2_user, h2_item)


# speedup vs baseline: 6.8012x; 1.0187x over previous
import functools

import jax
import jax.numpy as jnp
from jax import lax
from jax.experimental import pallas as pl
from jax.experimental.pallas import tpu as pltpu
from jax.experimental.pallas import tpu_sc as plsc

NC = 2
NS = 16
CH = 128
PBLK = 4
WBLK = 32
NBUF = 3


def _sc_mesh():
    return plsc.VectorSubcoreMesh(
        core_axis_name="c", subcore_axis_name="s", num_cores=NC, num_subcores=NS
    )


_SC_PARAMS = pltpu.CompilerParams(use_tc_tiling_on_sc=False)


def _grid_sizes(n_nodes, n_chunks):
    acc_rows = ((n_nodes + NS * 16) // (NS * 16)) * (NS * 16)
    stripe = acc_rows // NS
    last_rows = n_nodes - (NS - 1) * stripe
    assert 0 < last_rows <= stripe and last_rows % 8 == 0
    km = n_chunks // NS
    assert km % (2 * NBUF) == 0
    return acc_rows, stripe, last_rows, km


def _zchunk(stripe, width):
    for cand in (1, 2, 4, 8):
        if stripe % cand == 0 and (stripe // cand) * width * 4 <= 128 * 1024:
            return cand, stripe // cand
    raise ValueError(stripe)


def _segsum_body(n_nodes, km, stripe, last_rows, zrep,
                 xr, srcp4, dstp2d, zrows, out,
                 src_stage, dst_stage, rowbuf, acc, *sems):
    gsems = sems[:2 * NBUF]
    ssems = sems[2 * NBUF:]
    c = lax.axis_index("c")
    s = lax.axis_index("s")
    zr = zrows.shape[0]
    ngroups = km // NBUF

    for pass_ in range(PBLK // NC):
        p = pass_ * NC + c
        for j in range(zrep):
            pltpu.sync_copy(zrows, acc.at[pl.ds(s * stripe + j * zr, zr)])
        plsc.subcore_barrier()
        pass_base = (p * NS + s) * km
        dst_base = s * km

        def stage(g, par):
            pltpu.sync_copy(srcp4.at[pl.ds(pass_base + g * NBUF, NBUF)],
                            src_stage.at[par])
            pltpu.sync_copy(dstp2d.at[pl.ds(dst_base + g * NBUF, NBUF)],
                            dst_stage.at[par])

        def issue_gather(par, b):
            pltpu.async_copy(xr.at[src_stage.at[par, b]], rowbuf.at[par, b],
                             gsems[par * NBUF + b])

        def wait_gather(par, b):
            pltpu.make_async_copy(xr.at[src_stage.at[par, b]],
                                  rowbuf.at[par, b],
                                  gsems[par * NBUF + b]).wait()

        def issue_scatter(par, b):
            pltpu.async_copy(rowbuf.at[par, b], acc.at[dst_stage.at[par, b]],
                             ssems[par * NBUF + b], add=True)

        def wait_scatter(par, b):
            pltpu.make_async_copy(rowbuf.at[par, b],
                                  acc.at[dst_stage.at[par, b]],
                                  ssems[par * NBUF + b]).wait()

        stage(0, 0)
        for b in range(NBUF):
            issue_gather(0, b)

        def gbody(gg, carry):
            for par in range(2):
                u = 2 * gg + par
                npar = 1 - par

                @pl.when(u > 0)
                def _():
                    for b in range(NBUF):
                        wait_scatter(npar, b)

                @pl.when(u + 1 < ngroups)
                def _():
                    stage(u + 1, npar)
                    for b in range(NBUF):
                        issue_gather(npar, b)

                for b in range(NBUF):
                    wait_gather(par, b)
                    issue_scatter(par, b)
            return carry

        lax.fori_loop(0, ngroups // 2, gbody, 0)
        for b in range(NBUF):
            wait_scatter(1, b)
        plsc.subcore_barrier()
        row0 = s * stripe
        col0 = p * WBLK

        @pl.when(s < NS - 1)
        def _():
            pltpu.sync_copy(acc.at[pl.ds(row0, stripe)],
                            out.at[pl.ds(row0, stripe), pl.ds(col0, WBLK)])

        @pl.when(s == NS - 1)
        def _():
            pltpu.sync_copy(acc.at[pl.ds(row0, last_rows)],
                            out.at[pl.ds(row0, last_rows), pl.ds(col0, WBLK)])


def _make_segsum(n_nodes, n_chunks):
    acc_rows, stripe, last_rows, km = _grid_sizes(n_nodes, n_chunks)
    zrep, zr = _zchunk(stripe, WBLK)
    body = functools.partial(_segsum_body, n_nodes, km, stripe, last_rows, zrep)
    return pl.kernel(
        body,
        out_type=jax.ShapeDtypeStruct((n_nodes, 128), jnp.float32),
        mesh=_sc_mesh(),
        scratch_types=[
            pltpu.VMEM((2, NBUF, CH), jnp.int32),
            pltpu.VMEM((2, NBUF, CH), jnp.int32),
            pltpu.VMEM((2, NBUF, CH, WBLK), jnp.float32),
            pltpu.VMEM_SHARED((acc_rows, WBLK), jnp.float32),
        ] + [pltpu.SemaphoreType.DMA] * (4 * NBUF),
        compiler_params=_SC_PARAMS,
    ), zr


def _counts_body(n_nodes, km, stripe, last_rows, zrep,
                 dstc2d, ones, zrows, out, dst_stage, ones_v, zbuf, acc, *ssems):
    c = lax.axis_index("c")
    s = lax.axis_index("s")
    zr = zrows.shape[0]
    pltpu.sync_copy(ones, ones_v)
    pltpu.sync_copy(zrows, zbuf)
    for j in range(zrep):
        pltpu.sync_copy(zbuf, acc.at[pl.ds(s * stripe + j * zr, zr)])
    plsc.subcore_barrier()
    dst_base = (c * NS + s) * km
    ngroups = km // NBUF

    def stage(g, par):
        pltpu.sync_copy(dstc2d.at[pl.ds(dst_base + g * NBUF, NBUF)],
                        dst_stage.at[par])

    def issue_scatters(par):
        for b in range(NBUF):
            pltpu.async_copy(ones_v.at[b], acc.at[dst_stage.at[par, b]],
                             ssems[par * NBUF + b], add=True)

    def wait_scatters(par):
        for b in range(NBUF):
            pltpu.make_async_copy(ones_v.at[b], acc.at[dst_stage.at[par, b]],
                                  ssems[par * NBUF + b]).wait()

    stage(0, 0)
    issue_scatters(0)

    def gbody(gg, carry):
        for par in range(2):
            g = 2 * gg + par
            npar = 1 - par

            @pl.when(g + 1 < ngroups)
            def _():
                stage(g + 1, npar)
                issue_scatters(npar)

            wait_scatters(par)
        return carry

    lax.fori_loop(0, ngroups // 2, gbody, 0)
    plsc.subcore_barrier()
    row0 = s * stripe

    @pl.when(s < NS - 1)
    def _():
        pltpu.sync_copy(acc.at[pl.ds(row0, stripe)],
                        out.at[pl.ds(c * n_nodes + row0, stripe)])

    @pl.when(s == NS - 1)
    def _():
        pltpu.sync_copy(acc.at[pl.ds(row0, last_rows)],
                        out.at[pl.ds(c * n_nodes + row0, last_rows)])


def _make_counts(n_nodes, n_chunks):
    acc_rows, stripe, last_rows, km = _grid_sizes(n_nodes, n_chunks)
    zrep, zr = _zchunk(stripe, 16)
    body = functools.partial(_counts_body, n_nodes, km, stripe, last_rows, zrep)
    return pl.kernel(
        body,
        out_type=jax.ShapeDtypeStruct((2 * n_nodes, 16), jnp.float32),
        mesh=_sc_mesh(),
        scratch_types=[
            pltpu.VMEM((2, NBUF, CH), jnp.int32),
            pltpu.VMEM((NBUF, CH, 16), jnp.float32),
            pltpu.VMEM((zr, 16), jnp.float32),
            pltpu.VMEM_SHARED((acc_rows, 16), jnp.float32),
        ] + [pltpu.SemaphoreType.DMA] * (2 * NBUF),
        compiler_params=_SC_PARAMS,
    ), zr


def _tc_conv_kernel(relu, sum_ref, cnt_ref, xd_ref, wl_ref, bl_ref, wr_ref, out_ref):
    cnt = jnp.sum(cnt_ref[0], axis=1, keepdims=True) * (1.0 / 16.0)
    mean = sum_ref[...] / jnp.maximum(cnt, 1.0)
    dn = (((1,), (1,)), ((), ()))
    h = (lax.dot_general(mean, wl_ref[...], dn, preferred_element_type=jnp.float32)
         + bl_ref[...]
         + lax.dot_general(xd_ref[...], wr_ref[...], dn,
                           preferred_element_type=jnp.float32))
    if relu:
        h = jnp.maximum(h, 0.0)
    out_ref[...] = h


def _tc_conv(summed, cnts3, etype, x_dst, wl, bl, wr, relu):
    n = summed.shape[0]
    br = 1000 if n % 1000 == 0 else n
    return pl.pallas_call(
        functools.partial(_tc_conv_kernel, relu),
        grid=(n // br,),
        in_specs=[
            pl.BlockSpec((br, 128), lambda i: (i, 0)),
            pl.BlockSpec((1, br, 16), lambda i, t=etype: (t, i, 0)),
            pl.BlockSpec((br, 128), lambda i: (i, 0)),
            pl.BlockSpec((128, 128), lambda i: (0, 0)),
            pl.BlockSpec((1, 128), lambda i: (0, 0)),
            pl.BlockSpec((128, 128), lambda i: (0, 0)),
        ],
        out_specs=pl.BlockSpec((br, 128), lambda i: (i, 0)),
        out_shape=jax.ShapeDtypeStruct((n, 128), jnp.float32),
    )(summed, cnts3, x_dst, wl, bl.reshape(1, 128), wr)


def kernel(x_user, x_item, edge_index_user_item, edge_index_item_user,
           l1_ui_Wl, l1_ui_bl, l1_ui_Wr, l1_iu_Wl, l1_iu_bl, l1_iu_Wr,
           l2_ui_Wl, l2_ui_bl, l2_ui_Wr, l2_iu_Wl, l2_iu_bl, l2_iu_Wr):
    n = x_user.shape[0]
    e = edge_index_user_item.shape[1]
    gq = NS * 2 * NBUF
    g_chunks = ((e + CH - 1) // CH + gq - 1) // gq * gq
    ep = g_chunks * CH
    padn = ep - e
    i32 = jnp.int32
    trash = ((n + NS * 16) // (NS * 16)) * (NS * 16) - n
    pad_src = (jnp.arange(padn, dtype=i32) * 97) % n
    pad_dst = n + (jnp.arange(padn, dtype=i32) % max(trash, 1))

    def prep_edges(ei):
        src = jnp.concatenate([ei[0].astype(i32), pad_src])
        dst = jnp.concatenate([ei[1].astype(i32), pad_dst])
        srcr = src.reshape(g_chunks, CH)
        src4 = srcr[None] * PBLK + jnp.arange(PBLK, dtype=i32)[:, None, None]
        return src4.reshape(PBLK * g_chunks, CH), dst.reshape(g_chunks, CH)

    src4_ui, dst2d_ui = prep_edges(edge_index_user_item)
    src4_iu, dst2d_iu = prep_edges(edge_index_item_user)

    segsum, zr = _make_segsum(n, g_chunks)
    counts, zrc = _make_counts(n, g_chunks)

    zrows = jnp.zeros((zr, WBLK), jnp.float32)
    zrows16 = jnp.zeros((zrc, 16), jnp.float32)
    ones16 = jnp.ones((NBUF, CH, 16), jnp.float32)

    dstc2d = jnp.concatenate([dst2d_ui, dst2d_iu])
    cnts3 = counts(dstc2d, ones16, zrows16).reshape(2, n, 16)

    sum1_item = segsum(x_user.reshape(PBLK * n, WBLK), src4_ui, dst2d_ui, zrows)
    sum1_user = segsum(x_item.reshape(PBLK * n, WBLK), src4_iu, dst2d_iu, zrows)
    h1_item = _tc_conv(sum1_item, cnts3, 0, x_item, l1_ui_Wl, l1_ui_bl, l1_ui_Wr, True)
    h1_user = _tc_conv(sum1_user, cnts3, 1, x_user, l1_iu_Wl, l1_iu_bl, l1_iu_Wr, True)

    sum2_item = segsum(h1_user.reshape(PBLK * n, WBLK), src4_ui, dst2d_ui, zrows)
    sum2_user = segsum(h1_item.reshape(PBLK * n, WBLK), src4_iu, dst2d_iu, zrows)
    h2_item = _tc_conv(sum2_item, cnts3, 0, h1_item, l2_ui_Wl, l2_ui_bl, l2_ui_Wr, False)
    h2_user = _tc_conv(sum2_user, cnts3, 1, h1_user, l2_iu_Wl, l2_iu_bl, l2_iu_Wr, False)
    return (h2_user, h2_item)
